# baseline (speedup 1.0000x reference)
import functools

import jax
import jax.numpy as jnp
from jax import lax
from jax.experimental import pallas as pl
from jax.experimental.pallas import tpu as pltpu
from jax.experimental.pallas import tpu_sc as plsc

_NC, _NS, _LANES = 2, 16, 16
_NW = _NC * _NS

_NBUF = 4
_CHE = 8 * 1024
_UNROLL = 8


def _sc_add(nelems, seqelems):
    elems_per_w = nelems // _NW
    nchunk = elems_per_w // _CHE
    wpb = seqelems // elems_per_w

    mesh = plsc.VectorSubcoreMesh(core_axis_name="c", subcore_axis_name="s")

    @functools.partial(
        pl.kernel,
        mesh=mesh,
        out_type=jax.ShapeDtypeStruct((nelems,), jnp.float32),
        scratch_types=[
            pltpu.VMEM((_NBUF, _CHE), jnp.float32),
            pltpu.VMEM((_NBUF, _CHE), jnp.float32),
        ] + [pltpu.SemaphoreType.DMA] * (2 * _NBUF),
    )
    def body(x_hbm, p_hbm, o_hbm, xbuf, pbuf, *sems):
        lsems, ssems = sems[:_NBUF], sems[_NBUF:]
        c = lax.axis_index("c")
        s = lax.axis_index("s")
        wid = s * _NC + c
        base = wid * elems_per_w
        pbase = lax.rem(wid, wpb) * elems_per_w

        def issue_loads(k):
            b = k % _NBUF
            off = k * _CHE
            lx = pltpu.async_copy(x_hbm.at[pl.ds(base + off, _CHE)],
                                  xbuf.at[b], lsems[b])
            lp = pltpu.async_copy(p_hbm.at[pl.ds(pbase + off, _CHE)],
                                  pbuf.at[b], lsems[b])
            return lx, lp

        loads = {0: issue_loads(0), 1: issue_loads(1)}
        stores = {}
        for k in range(nchunk):
            b = k % _NBUF
            if k >= 2:
                stores.pop(k - 2).wait()
            if k + 2 < nchunk:
                loads[k + 2] = issue_loads(k + 2)
            lx, lp = loads.pop(k)
            lx.wait()
            lp.wait()

            def cbody(i, _, b=b):
                o = i * (_UNROLL * _LANES)
                for j in range(_UNROLL):
                    sl = pl.ds(o + j * _LANES, _LANES)
                    xbuf[b, sl] = xbuf[b, sl] + pbuf[b, sl]
                return 0

            lax.fori_loop(0, _CHE // (_UNROLL * _LANES), cbody, 0)
            stores[k] = pltpu.async_copy(
                xbuf.at[b], o_hbm.at[pl.ds(base + k * _CHE, _CHE)], ssems[b])
        for k in sorted(stores):
            stores.pop(k).wait()

    return body


def kernel(x, pos_table):
    B, L, D = x.shape
    xf = x.reshape(B * L * D)
    pf = pos_table.reshape(pos_table.shape[0] * D)
    out = _sc_add(B * L * D, L * D)(xf, pf)
    return out.reshape(B, L, D)


# speedup vs baseline: 1.0038x; 1.0038x over previous
import functools

import jax
import jax.numpy as jnp
from jax import lax
from jax.experimental import pallas as pl
from jax.experimental.pallas import tpu as pltpu
from jax.experimental.pallas import tpu_sc as plsc

_NC, _NS, _LANES = 2, 16, 16
_NW = _NC * _NS

_NBUF = 4
_CHE = 8 * 1024
_UNROLL = 8


def _sc_add(nelems, seqelems):
    elems_per_w = nelems // _NW
    nchunk = elems_per_w // _CHE
    wpb = seqelems // elems_per_w

    mesh = plsc.VectorSubcoreMesh(core_axis_name="c", subcore_axis_name="s")

    @functools.partial(
        pl.kernel,
        mesh=mesh,
        out_type=jax.ShapeDtypeStruct((nelems,), jnp.float32),
        scratch_types=[
            pltpu.VMEM((_NBUF, _CHE), jnp.float32),
            pltpu.VMEM((_NBUF, _CHE), jnp.float32),
        ] + [pltpu.SemaphoreType.DMA] * (2 * _NBUF),
    )
    def body(x_hbm, p_hbm, o_hbm, xbuf, pbuf, *sems):
        lsems, ssems = sems[:_NBUF], sems[_NBUF:]
        c = lax.axis_index("c")
        s = lax.axis_index("s")
        wid = s * _NC + c
        base = wid * elems_per_w
        pbase = lax.rem(wid, wpb) * elems_per_w

        def issue_loads(k):
            b = k % _NBUF
            off = k * _CHE
            lx = pltpu.async_copy(x_hbm.at[pl.ds(base + off, _CHE)],
                                  xbuf.at[b], lsems[b])
            lp = pltpu.async_copy(p_hbm.at[pl.ds(pbase + off, _CHE)],
                                  pbuf.at[b], lsems[b])
            return lx, lp

        loads = {0: issue_loads(0), 1: issue_loads(1)}
        stores = {}
        for k in range(nchunk):
            b = k % _NBUF
            if k >= 2:
                stores.pop(k - 2).wait()
            if k + 2 < nchunk:
                loads[k + 2] = issue_loads(k + 2)
            lx, lp = loads.pop(k)
            lx.wait()
            lp.wait()

            @plsc.parallel_loop(0, _CHE, step=_LANES, unroll=_UNROLL)
            def cbody(o, b=b):
                sl = pl.ds(o, _LANES)
                xbuf[b, sl] = xbuf[b, sl] + pbuf[b, sl]
            stores[k] = pltpu.async_copy(
                xbuf.at[b], o_hbm.at[pl.ds(base + k * _CHE, _CHE)], ssems[b])
        for k in sorted(stores):
            stores.pop(k).wait()

    return body


def kernel(x, pos_table):
    B, L, D = x.shape
    xf = x.reshape(B * L * D)
    pf = pos_table.reshape(pos_table.shape[0] * D)
    out = _sc_add(B * L * D, L * D)(xf, pf)
    return out.reshape(B, L, D)


# speedup vs baseline: 2.8570x; 2.8460x over previous
import functools

import jax
import jax.numpy as jnp
from jax import lax
from jax.experimental import pallas as pl
from jax.experimental.pallas import tpu as pltpu
from jax.experimental.pallas import tpu_sc as plsc

_NC, _NS, _LANES = 2, 16, 16
_NW = _NC * _NS

_NBUF = 3
_CH = 16
_UNROLL = 8


def _sc_add(nrows, nseq, d):
    rows_per_w = nrows // _NW
    nchunk = rows_per_w // _CH
    wpb = nseq // rows_per_w

    mesh = plsc.VectorSubcoreMesh(core_axis_name="c", subcore_axis_name="s")

    @functools.partial(
        pl.kernel,
        mesh=mesh,
        out_type=jax.ShapeDtypeStruct((nrows, d), jnp.float32),
        scratch_types=[
            pltpu.VMEM((_NBUF, _CH, d), jnp.float32),
            pltpu.VMEM((_NBUF, _CH, d), jnp.float32),
        ] + [pltpu.SemaphoreType.DMA] * (2 * _NBUF),
    )
    def body(x_hbm, p_hbm, o_hbm, xbuf, pbuf, *sems):
        lsems, ssems = sems[:_NBUF], sems[_NBUF:]
        c = lax.axis_index("c")
        s = lax.axis_index("s")
        wid = s * _NC + c
        base = wid * rows_per_w
        pbase = lax.rem(wid, wpb) * rows_per_w

        def issue_loads(k):
            b = k % _NBUF
            off = k * _CH
            lx = pltpu.async_copy(x_hbm.at[pl.ds(base + off, _CH)],
                                  xbuf.at[b], lsems[b])
            lp = pltpu.async_copy(p_hbm.at[pl.ds(pbase + off, _CH)],
                                  pbuf.at[b], lsems[b])
            return lx, lp

        loads = {0: issue_loads(0)}
        stores = {}
        for k in range(nchunk):
            b = k % _NBUF
            if k >= 2:
                stores.pop(k - 2).wait()
            if k + 1 < nchunk:
                loads[k + 1] = issue_loads(k + 1)
            lx, lp = loads.pop(k)
            lx.wait()
            lp.wait()

            def rbody(r, _, b=b):
                @plsc.parallel_loop(0, d, step=_LANES, unroll=_UNROLL)
                def cbody(o):
                    sl = pl.ds(o, _LANES)
                    xbuf[b, r, sl] = xbuf[b, r, sl] + pbuf[b, r, sl]
                return 0

            lax.fori_loop(0, _CH, rbody, 0)
            stores[k] = pltpu.async_copy(
                xbuf.at[b], o_hbm.at[pl.ds(base + k * _CH, _CH)], ssems[b])
        for k in sorted(stores):
            stores.pop(k).wait()

    return body


def kernel(x, pos_table):
    B, L, D = x.shape
    xf = x.reshape(B * L, D)
    out = _sc_add(B * L, L, D)(xf, pos_table)
    return out.reshape(B, L, D)


# speedup vs baseline: 3.4360x; 1.2027x over previous
import functools

import jax
import jax.numpy as jnp
from jax import lax
from jax.experimental import pallas as pl
from jax.experimental.pallas import tpu as pltpu
from jax.experimental.pallas import tpu_sc as plsc

_NC, _NS, _LANES = 2, 16, 16
_NW = _NC * _NS

_NBUF = 3
_NPB = 3
_CH = 16
_UNROLL = 8


def _sc_add(nbatch, nseq, d):
    seq_per_w = nseq // _NW
    nchunk = seq_per_w // _CH
    nunits = nchunk * nbatch

    mesh = plsc.VectorSubcoreMesh(core_axis_name="c", subcore_axis_name="s")

    @functools.partial(
        pl.kernel,
        mesh=mesh,
        out_type=jax.ShapeDtypeStruct((nbatch * nseq, d), jnp.float32),
        scratch_types=[
            pltpu.VMEM((_NBUF, _CH, d), jnp.float32),
            pltpu.VMEM((_NPB, _CH, d), jnp.float32),
        ] + [pltpu.SemaphoreType.DMA] * (_NBUF + _NPB + _NBUF),
    )
    def body(x_hbm, p_hbm, o_hbm, xbuf, pbuf, *sems):
        lsems = sems[:_NBUF]
        psems = sems[_NBUF:_NBUF + _NPB]
        ssems = sems[_NBUF + _NPB:]
        c = lax.axis_index("c")
        s = lax.axis_index("s")
        wid = s * _NC + c
        base = wid * seq_per_w

        def issue_x(u):
            k, b = divmod(u, nbatch)
            buf = u % _NBUF
            roff = b * nseq + base + k * _CH
            return pltpu.async_copy(x_hbm.at[pl.ds(roff, _CH)],
                                    xbuf.at[buf], lsems[buf])

        def issue_p(k):
            pb = k % _NPB
            return pltpu.async_copy(p_hbm.at[pl.ds(base + k * _CH, _CH)],
                                    pbuf.at[pb], psems[pb])

        ploads = {0: issue_p(0)}
        xloads = {0: issue_x(0)}
        stores = {}
        for k in range(nchunk):
            pb = k % _NPB
            if k + 1 < nchunk:
                ploads[k + 1] = issue_p(k + 1)
            ploads.pop(k).wait()
            for b in range(nbatch):
                u = k * nbatch + b
                buf = u % _NBUF
                if u >= 2:
                    stores.pop(u - 2).wait()
                if u + 1 < nunits:
                    xloads[u + 1] = issue_x(u + 1)
                xloads.pop(u).wait()

                def rbody(r, _, buf=buf, pb=pb):
                    @plsc.parallel_loop(0, d, step=_LANES, unroll=_UNROLL)
                    def cbody(o):
                        sl = pl.ds(o, _LANES)
                        xbuf[buf, r, sl] = xbuf[buf, r, sl] + pbuf[pb, r, sl]
                    return 0

                lax.fori_loop(0, _CH, rbody, 0)
                roff = b * nseq + base + k * _CH
                stores[u] = pltpu.async_copy(
                    xbuf.at[buf], o_hbm.at[pl.ds(roff, _CH)], ssems[buf])
        for u in sorted(stores):
            stores.pop(u).wait()

    return body


def kernel(x, pos_table):
    B, L, D = x.shape
    xf = x.reshape(B * L, D)
    out = _sc_add(B, L, D)(xf, pos_table)
    return out.reshape(B, L, D)


# speedup vs baseline: 3.6553x; 1.0638x over previous
import functools

import jax
import jax.numpy as jnp
from jax import lax
from jax.experimental import pallas as pl
from jax.experimental.pallas import tpu as pltpu
from jax.experimental.pallas import tpu_sc as plsc

_NC, _NS, _LANES = 2, 16, 16
_NW = _NC * _NS

_NBUF = 4
_NPB = 2
_CH = 16
_UNROLL = 8
_XPF = 2


def _sc_add(nbatch, nseq, d):
    seq_per_w = nseq // _NW
    nchunk = seq_per_w // _CH
    nunits = nchunk * nbatch
    assert d & (d - 1) == 0
    dshift = d.bit_length() - 1

    mesh = plsc.VectorSubcoreMesh(core_axis_name="c", subcore_axis_name="s")

    @functools.partial(
        pl.kernel,
        mesh=mesh,
        out_type=jax.ShapeDtypeStruct((nbatch * nseq, d), jnp.float32),
        scratch_types=[
            pltpu.VMEM((_NBUF, _CH, d), jnp.float32),
            pltpu.VMEM((_NPB, _CH, d), jnp.float32),
        ] + [pltpu.SemaphoreType.DMA] * (_NBUF + _NPB + _NBUF),
    )
    def body(x_hbm, p_hbm, o_hbm, xbuf, pbuf, *sems):
        lsems = sems[:_NBUF]
        psems = sems[_NBUF:_NBUF + _NPB]
        ssems = sems[_NBUF + _NPB:]
        c = lax.axis_index("c")
        s = lax.axis_index("s")
        wid = s * _NC + c
        base = wid * seq_per_w

        def issue_x(u):
            k, b = divmod(u, nbatch)
            buf = u % _NBUF
            roff = b * nseq + base + k * _CH
            return pltpu.async_copy(x_hbm.at[pl.ds(roff, _CH)],
                                    xbuf.at[buf], lsems[buf])

        def issue_p(k):
            pb = k % _NPB
            return pltpu.async_copy(p_hbm.at[pl.ds(base + k * _CH, _CH)],
                                    pbuf.at[pb], psems[pb])

        ploads = {0: issue_p(0)}
        xloads = {u: issue_x(u) for u in range(min(_XPF, nunits))}
        stores = {}
        for k in range(nchunk):
            pb = k % _NPB
            if k + 1 < nchunk:
                ploads[k + 1] = issue_p(k + 1)
            ploads.pop(k).wait()
            for b in range(nbatch):
                u = k * nbatch + b
                buf = u % _NBUF
                if u >= _NBUF - _XPF:
                    stores.pop(u - (_NBUF - _XPF)).wait()
                if u + _XPF < nunits:
                    xloads[u + _XPF] = issue_x(u + _XPF)
                xloads.pop(u).wait()

                @plsc.parallel_loop(0, _CH * d, step=_LANES, unroll=_UNROLL)
                def cbody(o, buf=buf, pb=pb):
                    r = o >> dshift
                    sl = pl.ds(pl.multiple_of(o & (d - 1), _LANES), _LANES)
                    xbuf[buf, r, sl] = xbuf[buf, r, sl] + pbuf[pb, r, sl]

                roff = b * nseq + base + k * _CH
                stores[u] = pltpu.async_copy(
                    xbuf.at[buf], o_hbm.at[pl.ds(roff, _CH)], ssems[buf])
        for u in sorted(stores):
            stores.pop(u).wait()

    return body


def kernel(x, pos_table):
    B, L, D = x.shape
    xf = x.reshape(B * L, D)
    out = _sc_add(B, L, D)(xf, pos_table)
    return out.reshape(B, L, D)
